# baseline (speedup 1.0000x reference)
import jax
import jax.numpy as jnp
from jax.experimental import pallas as pl
from jax.experimental.pallas import tpu as pltpu

_DIM = 1024
_BATCH = 32
_DBLK = 32


def _tc_body(t_ref, table_ref, pos_ref, out_ref, rows_ref):
    i = pl.program_id(0)

    @pl.when(i == 0)
    def _gather():
        def body(b, carry):
            rows_ref[pl.ds(b, 1), :] = table_ref[pl.ds(t_ref[b], 1), :]
            return carry

        jax.lax.fori_loop(0, _BATCH, body, 0)

    pos_vals = pos_ref[0, :, 0, 0]
    rows = rows_ref[:, :]
    out_ref[0] = pos_vals[:, None, None] + rows[None, :, :]


def kernel(t, table, pos_emb):
    t = t.astype(jnp.int32)
    grid = (_DIM // _DBLK,)
    return pl.pallas_call(
        _tc_body,
        grid_spec=pltpu.PrefetchScalarGridSpec(
            num_scalar_prefetch=1,
            grid=grid,
            in_specs=[
                pl.BlockSpec((_DIM, _DIM), lambda i, t_pref: (0, 0)),
                pl.BlockSpec((1, _DBLK, 1, 1), lambda i, t_pref: (0, i, 0, 0)),
            ],
            out_specs=pl.BlockSpec(
                (1, _DBLK, _BATCH, _DIM), lambda i, t_pref: (0, i, 0, 0)
            ),
            scratch_shapes=[pltpu.VMEM((_BATCH, _DIM), jnp.float32)],
        ),
        out_shape=jax.ShapeDtypeStruct((1, _DIM, _BATCH, _DIM), jnp.float32),
    )(t, table, pos_emb)
